# baseline (speedup 1.0000x reference)
import jax
import jax.numpy as jnp
from jax import lax
from jax.experimental import pallas as pl
from jax.experimental.pallas import tpu as pltpu
from jax.experimental.pallas import tpu_sc as plsc

_B = 16384
_C = 5
_D = 64
_NC = 2
_NS = 16
_NW = _NC * _NS
_BPW = _B // _NW
_CB = 128
_NCHUNK = _BPW // _CB
_PAIRS = _CB * _C


def _sc_body(tgt_hbm, ctx_hbm, ttab_hbm, ctab_hbm, out_hbm,
             tidx, cidx, trows, crows, outbuf, sem):
    wid = lax.axis_index("s") * _NC + lax.axis_index("c")
    base = wid * _BPW
    for chunk in range(_NCHUNK):
        b0 = base + chunk * _CB
        pltpu.sync_copy(tgt_hbm.at[pl.ds(b0, _CB)], tidx)
        for j in range(_C):
            pltpu.sync_copy(ctx_hbm.at[pl.ds(b0 * _C + j * _CB, _CB)],
                            cidx.at[j])
        pltpu.async_copy(ttab_hbm.at[tidx], trows, sem).wait()
        for j in range(_C):
            pltpu.async_copy(ctab_hbm.at[cidx.at[j]],
                             crows.at[pl.ds(j * _CB, _CB)], sem).wait()

        def zero(i, carry):
            outbuf[pl.ds(i * 16, 16)] = jnp.zeros((16,), jnp.float32)
            return carry

        lax.fori_loop(0, _PAIRS // 16, zero, 0)

        def body(b, carry):
            w = [trows[b, pl.ds(16 * j, 16)] for j in range(_D // 16)]
            for c in range(_C):
                row = b * _C + c
                acc = w[0] * crows[row, pl.ds(0, 16)]
                for j in range(1, _D // 16):
                    acc = acc + w[j] * crows[row, pl.ds(16 * j, 16)]
                idx = jnp.full((16,), row, jnp.int32)
                plsc.addupdate_scatter(outbuf, [idx], acc)
            return carry

        lax.fori_loop(0, _CB, body, 0)
        pltpu.sync_copy(outbuf, out_hbm.at[pl.ds(b0 * _C, _PAIRS)])


@jax.jit
def kernel(target, context, target_table, context_table):
    ctx_flat = context.reshape(-1).astype(jnp.int32)
    tgt = target.astype(jnp.int32)
    mesh = plsc.VectorSubcoreMesh(core_axis_name="c", subcore_axis_name="s",
                                  num_cores=_NC, num_subcores=_NS)
    out_flat = pl.kernel(
        _sc_body,
        out_type=jax.ShapeDtypeStruct((_B * _C,), jnp.float32),
        mesh=mesh,
        compiler_params=pltpu.CompilerParams(needs_layout_passes=False,
                                             use_tc_tiling_on_sc=False),
        scratch_types=[
            pltpu.VMEM((_CB,), jnp.int32),
            pltpu.VMEM((_C, _CB), jnp.int32),
            pltpu.VMEM((_CB, _D), jnp.float32),
            pltpu.VMEM((_PAIRS, _D), jnp.float32),
            pltpu.VMEM((_PAIRS,), jnp.float32),
            pltpu.SemaphoreType.DMA,
        ],
    )(tgt, ctx_flat, target_table, context_table)
    return out_flat.reshape(_B, _C)


# speedup vs baseline: 1.4872x; 1.4872x over previous
import jax
import jax.numpy as jnp
from jax import lax
from jax.experimental import pallas as pl
from jax.experimental.pallas import tpu as pltpu
from jax.experimental.pallas import tpu_sc as plsc

_B = 16384
_C = 5
_D = 64
_NC = 2
_NS = 16
_NW = _NC * _NS
_BPW = _B // _NW
_CB = 128
_NCHUNK = _BPW // _CB
_PAIRS = _CB * _C


def _sc_body(tgt_hbm, ctx_hbm, ttab_hbm, ctab_hbm, out_hbm,
             tidx, cidx, trows, crows, outbuf, sem):
    wid = lax.axis_index("s") * _NC + lax.axis_index("c")
    base = wid * _BPW
    for chunk in range(_NCHUNK):
        b0 = base + chunk * _CB
        pltpu.sync_copy(tgt_hbm.at[pl.ds(b0, _CB)], tidx)
        pltpu.sync_copy(ctx_hbm.at[pl.ds(b0 * _C, _PAIRS)], cidx)

        def fire_t(g, carry):
            v = tidx[pl.ds(g * 16, 16)]
            for i in range(16):
                pltpu.make_async_copy(ttab_hbm.at[pl.ds(v[i], 1)],
                                      trows.at[pl.ds(g * 16 + i, 1)],
                                      sem).start()
            return carry

        lax.fori_loop(0, _CB // 16, fire_t, 0)

        def fire_c(g, carry):
            v = cidx[pl.ds(g * 16, 16)]
            for i in range(16):
                pltpu.make_async_copy(ctab_hbm.at[pl.ds(v[i], 1)],
                                      crows.at[pl.ds(g * 16 + i, 1)],
                                      sem).start()
            return carry

        lax.fori_loop(0, _PAIRS // 16, fire_c, 0)

        def drain_t(k, carry):
            pltpu.make_async_copy(ttab_hbm.at[pl.ds(0, 1)],
                                  trows.at[pl.ds(k, 1)], sem).wait()
            return carry

        lax.fori_loop(0, _CB, drain_t, 0)

        def drain_c(k, carry):
            pltpu.make_async_copy(ctab_hbm.at[pl.ds(0, 1)],
                                  crows.at[pl.ds(k, 1)], sem).wait()
            return carry

        lax.fori_loop(0, _PAIRS, drain_c, 0)

        def zero(i, carry):
            outbuf[pl.ds(i * 16, 16)] = jnp.zeros((16,), jnp.float32)
            return carry

        lax.fori_loop(0, _PAIRS // 16, zero, 0)

        def body(b, carry):
            w = [trows[b, pl.ds(16 * j, 16)] for j in range(_D // 16)]
            for c in range(_C):
                row = b * _C + c
                acc = w[0] * crows[row, pl.ds(0, 16)]
                for j in range(1, _D // 16):
                    acc = acc + w[j] * crows[row, pl.ds(16 * j, 16)]
                idx = jnp.full((16,), row, jnp.int32)
                plsc.addupdate_scatter(outbuf, [idx], acc)
            return carry

        lax.fori_loop(0, _CB, body, 0)
        pltpu.sync_copy(outbuf, out_hbm.at[pl.ds(b0 * _C, _PAIRS)])


@jax.jit
def kernel(target, context, target_table, context_table):
    tgt = target.astype(jnp.int32)
    ctx = context.reshape(-1).astype(jnp.int32)
    mesh = plsc.VectorSubcoreMesh(core_axis_name="c", subcore_axis_name="s",
                                  num_cores=_NC, num_subcores=_NS)
    out_flat = pl.kernel(
        _sc_body,
        out_type=jax.ShapeDtypeStruct((_B * _C,), jnp.float32),
        mesh=mesh,
        compiler_params=pltpu.CompilerParams(needs_layout_passes=False,
                                             use_tc_tiling_on_sc=True),
        scratch_types=[
            pltpu.VMEM((_CB,), jnp.int32),
            pltpu.VMEM((_PAIRS,), jnp.int32),
            pltpu.VMEM((_CB, _D), jnp.float32),
            pltpu.VMEM((_PAIRS, _D), jnp.float32),
            pltpu.VMEM((_PAIRS,), jnp.float32),
            pltpu.SemaphoreType.DMA,
        ],
    )(tgt, ctx, target_table, context_table)
    return out_flat.reshape(_B, _C)
